# baseline (speedup 1.0000x reference)
import functools

import jax
import jax.numpy as jnp
from jax import lax
from jax.experimental import pallas as pl
from jax.experimental.pallas import tpu as pltpu
from jax.experimental.pallas import tpu_sc as plsc


def _sc_gather(inputs, table):
    B = inputs.shape[0]
    V, D = table.shape
    info = plsc.get_sparse_core_info()
    nw = info.num_cores * info.num_subcores
    b_per_w = B // nw
    mesh = plsc.VectorSubcoreMesh(core_axis_name="c", subcore_axis_name="s")

    @functools.partial(
        pl.kernel,
        mesh=mesh,
        out_type=jax.ShapeDtypeStruct((B, D), jnp.float32),
        scratch_types=[
            pltpu.VMEM((b_per_w,), jnp.int32),
            pltpu.VMEM((b_per_w, D), jnp.float32),
            pltpu.SemaphoreType.DMA,
        ],
    )
    def gather_kernel(idx_hbm, table_hbm, out_hbm, idx_v, rows_v, sem):
        wid = lax.axis_index("s") * info.num_cores + lax.axis_index("c")
        base = wid * b_per_w
        pltpu.sync_copy(idx_hbm.at[pl.ds(base, b_per_w)], idx_v)
        pltpu.async_copy(table_hbm.at[idx_v], rows_v, sem).wait()
        pltpu.sync_copy(rows_v, out_hbm.at[pl.ds(base, b_per_w)])

    return gather_kernel(inputs, table)


def _tc_project(emb, W, b, block_n):
    B, D = emb.shape
    V = W.shape[1]
    nn = pl.cdiv(V, block_n)
    b2 = b.reshape(1, V)

    def mm_kernel(emb_ref, w_ref, b_ref, out_ref):
        out_ref[...] = (
            jnp.dot(emb_ref[...], w_ref[...], preferred_element_type=jnp.float32)
            + b_ref[...]
        )

    return pl.pallas_call(
        mm_kernel,
        grid=(nn,),
        in_specs=[
            pl.BlockSpec((B, D), lambda j: (0, 0)),
            pl.BlockSpec((D, block_n), lambda j: (0, j)),
            pl.BlockSpec((1, block_n), lambda j: (0, j)),
        ],
        out_specs=pl.BlockSpec((B, block_n), lambda j: (0, j)),
        out_shape=jax.ShapeDtypeStruct((B, V), jnp.float32),
    )(emb, W, b2)


def kernel(inputs, table, W, b):
    emb = _sc_gather(inputs, table)
    return _tc_project(emb, W, b, block_n=512)


# speedup vs baseline: 1.0098x; 1.0098x over previous
import functools

import jax
import jax.numpy as jnp
from jax import lax
from jax.experimental import pallas as pl
from jax.experimental.pallas import tpu as pltpu
from jax.experimental.pallas import tpu_sc as plsc


def _sc_gather(inputs, table):
    B = inputs.shape[0]
    V, D = table.shape
    info = plsc.get_sparse_core_info()
    nw = info.num_cores * info.num_subcores
    b_per_w = B // nw
    mesh = plsc.VectorSubcoreMesh(core_axis_name="c", subcore_axis_name="s")

    @functools.partial(
        pl.kernel,
        mesh=mesh,
        out_type=jax.ShapeDtypeStruct((B, D), jnp.float32),
        scratch_types=[
            pltpu.VMEM((b_per_w,), jnp.int32),
            pltpu.VMEM((b_per_w, D), jnp.float32),
            pltpu.SemaphoreType.DMA,
        ],
    )
    def gather_kernel(idx_hbm, table_hbm, out_hbm, idx_v, rows_v, sem):
        wid = lax.axis_index("s") * info.num_cores + lax.axis_index("c")
        base = wid * b_per_w
        pltpu.sync_copy(idx_hbm.at[pl.ds(base, b_per_w)], idx_v)
        pltpu.async_copy(table_hbm.at[idx_v], rows_v, sem).wait()
        pltpu.sync_copy(rows_v, out_hbm.at[pl.ds(base, b_per_w)])

    return gather_kernel(inputs, table)


def _tc_project(emb, W, b, block_n, nbuf):
    B, D = emb.shape
    V = W.shape[1]
    grid = pl.cdiv(V, block_n)
    rem = V - (grid - 1) * block_n
    edge_j = grid - 1
    b2 = b.reshape(1, V)

    n_full = V // block_n

    def mm_kernel(emb_ref, w_ref, b_ref, out_hbm, bufs, sems):
        j = pl.program_id(0)
        slot = jax.lax.rem(j, nbuf)
        acc = (
            jnp.dot(emb_ref[...], w_ref[...], preferred_element_type=jnp.float32)
            + b_ref[...]
        )
        for k in range(nbuf):
            @pl.when(jnp.logical_and(slot == k, j >= nbuf))
            def _():
                pltpu.make_async_copy(
                    bufs.at[k], out_hbm.at[:, pl.ds(0, block_n)], sems.at[k]
                ).wait()

            @pl.when(slot == k)
            def _():
                bufs[k] = acc
                pltpu.make_async_copy(
                    bufs.at[k],
                    out_hbm.at[:, pl.ds(j * block_n, block_n)],
                    sems.at[k],
                ).start()

        @pl.when(j == n_full - 1)
        def _():
            for k2 in range(min(nbuf, n_full)):
                pltpu.make_async_copy(
                    bufs.at[k2], out_hbm.at[:, pl.ds(0, block_n)], sems.at[k2]
                ).wait()

    partial = pl.pallas_call(
        mm_kernel,
        grid=(n_full,),
        in_specs=[
            pl.BlockSpec((B, D), lambda j: (0, 0)),
            pl.BlockSpec((D, block_n), lambda j: (0, j)),
            pl.BlockSpec((1, block_n), lambda j: (0, j)),
        ],
        out_specs=pl.BlockSpec(memory_space=pl.ANY),
        out_shape=jax.ShapeDtypeStruct((B, V), jnp.float32),
        scratch_shapes=[
            pltpu.VMEM((nbuf, B, block_n), jnp.float32),
            pltpu.SemaphoreType.DMA((nbuf,)),
        ],
    )(emb, W, b2)

    if n_full * block_n == V:
        return partial

    def edge_kernel(emb_ref, w_ref, b_ref, full_ref, out_ref):
        del full_ref
        out_ref[...] = (
            jnp.dot(emb_ref[...], w_ref[...], preferred_element_type=jnp.float32)
            + b_ref[...]
        )

    return pl.pallas_call(
        edge_kernel,
        grid=(1,),
        in_specs=[
            pl.BlockSpec((B, D), lambda j: (0, 0)),
            pl.BlockSpec((D, block_n), lambda j: (0, n_full)),
            pl.BlockSpec((1, block_n), lambda j: (0, n_full)),
            pl.BlockSpec(memory_space=pl.ANY),
        ],
        out_specs=pl.BlockSpec((B, block_n), lambda j: (0, n_full)),
        out_shape=jax.ShapeDtypeStruct((B, V), jnp.float32),
        input_output_aliases={3: 0},
    )(emb, W, b2, partial)


def kernel(inputs, table, W, b):
    emb = _sc_gather(inputs, table)
    return _tc_project(emb, W, b, block_n=512, nbuf=4)
